# baseline (speedup 1.0000x reference)
import jax
import jax.numpy as jnp
import numpy as np
from jax.experimental import pallas as pl
from jax.experimental.pallas import tpu as pltpu

B, PH, PW, D = 8, 16, 16, 1024
P = PH * PW
N = B * P
NPAIR = B * (B - 1) // 2


def _prep(f_ref, xs, hc, hr):
    x = f_ref[...].reshape(N, D)
    mu = jnp.mean(x, axis=-1, keepdims=True)
    var = jnp.mean((x - mu) * (x - mu), axis=-1, keepdims=True)
    xn = (x - mu) / jnp.sqrt(var + 1e-6)

    io = jax.lax.broadcasted_iota(jnp.int32, (N, 1), 0)
    pm = io % P
    ph = pm // PW
    pw = pm % PW
    z16 = jnp.zeros((16, D), jnp.float32)
    z1 = jnp.zeros((1, D), jnp.float32)
    up = jnp.concatenate([xn[16:], z16], axis=0)
    dn = jnp.concatenate([z16, xn[:-16]], axis=0)
    vs = xn + jnp.where(ph < PH - 1, up, 0.0) + jnp.where(ph > 0, dn, 0.0)
    lf = jnp.concatenate([vs[1:], z1], axis=0)
    rt = jnp.concatenate([z1, vs[:-1]], axis=0)
    hs = vs + jnp.where(pw < PW - 1, lf, 0.0) + jnp.where(pw > 0, rt, 0.0)
    cv = 3.0 - (ph == 0) - (ph == PH - 1)
    ch = 3.0 - (pw == 0) - (pw == PW - 1)
    pooled = hs / (cv * ch)

    xs[0] = xn.astype(jnp.bfloat16)
    xs[1] = pooled.astype(jnp.bfloat16)
    ones_row = jnp.ones((1, D), jnp.float32)
    for r in range(2):
        a = [xn, pooled][r]
        hc[r] = 0.5 * jnp.sum(a * a, axis=1, keepdims=True)
        for b in range(B):
            zb = a[b * P:(b + 1) * P]
            hr[r, pl.ds(b, 1)] = 0.5 * jax.lax.dot_general(
                ones_row, zb * zb, (((1,), (1,)), ((), ())),
                preferred_element_type=jnp.float32)


def _merge_top2(a1, a2, b1, b2):
    m1 = jnp.minimum(a1, b1)
    m2 = jnp.minimum(jnp.maximum(a1, b1), jnp.minimum(a2, b2))
    return m1, m2


def _issue_matmul(p, ii_ref, jj_ref, r, xs, gw):
    i = ii_ref[p]
    j = jj_ref[p]
    xq = xs[r, pl.ds(i * P, P)]
    xk = xs[r, pl.ds(j * P, P)]
    gw[...] = jax.lax.dot_general(xk, xq, (((1,), (1,)), ((), ())),
                                  preferred_element_type=jnp.float32)


def _epilogue(q, ii_ref, jj_ref, r, gr, hc, hr, m1r, m2r, m1c, m2c):
    i = ii_ref[q]
    j = jj_ref[q]
    hq = hr[r, pl.ds(i, 1)]
    hk = hc[r, pl.ds(j * P, P)]
    h2 = (hk - gr[...]) + hq
    dq = jnp.min(h2, axis=0, keepdims=True)
    dk = jnp.min(h2, axis=1, keepdims=True)

    rows = jax.lax.broadcasted_iota(jnp.int32, (B, P), 0)
    urow = rows == i
    dqb = jnp.broadcast_to(dq, (B, P))
    o1 = m1r[...]
    m1r[...] = jnp.where(urow, jnp.minimum(o1, dqb), o1)
    m2r[...] = jnp.where(urow & (dqb < o1), o1,
                         jnp.where(urow, jnp.minimum(m2r[...], dqb), m2r[...]))

    cols = jax.lax.broadcasted_iota(jnp.int32, (P, B), 1)
    ucol = cols == j
    dkb = jnp.broadcast_to(dk, (P, B))
    c1 = m1c[...]
    m1c[...] = jnp.where(ucol, jnp.minimum(c1, dkb), c1)
    m2c[...] = jnp.where(ucol & (dkb < c1), c1,
                         jnp.where(ucol, jnp.minimum(m2c[...], dkb), m2c[...]))


def _fused_kernel(ii_ref, jj_ref, f_ref, ps_ref, sc_ref,
                  xs, hc, hr, ga, gb, m1r, m2r, m1c, m2c, acc):
    r = pl.program_id(0)
    p = pl.program_id(1)

    @pl.when((r == 0) & (p == 0))
    def _():
        _prep(f_ref, xs, hc, hr)

    @pl.when(p == 0)
    def _():
        m1r[...] = jnp.full((B, P), jnp.inf, jnp.float32)
        m2r[...] = jnp.full((B, P), jnp.inf, jnp.float32)
        m1c[...] = jnp.full((P, B), jnp.inf, jnp.float32)
        m2c[...] = jnp.full((P, B), jnp.inf, jnp.float32)

    even = p % 2 == 0

    @pl.when((p < NPAIR) & even)
    def _():
        _issue_matmul(p, ii_ref, jj_ref, r, xs, ga)

    @pl.when((p < NPAIR) & jnp.logical_not(even))
    def _():
        _issue_matmul(p, ii_ref, jj_ref, r, xs, gb)

    @pl.when((p > 0) & jnp.logical_not(even))
    def _():
        _epilogue(p - 1, ii_ref, jj_ref, r, ga, hc, hr, m1r, m2r, m1c, m2c)

    @pl.when((p > 0) & even)
    def _():
        _epilogue(p - 1, ii_ref, jj_ref, r, gb, hc, hr, m1r, m2r, m1c, m2c)

    @pl.when(p == NPAIR)
    def _():
        t1 = m1c[...].T
        t2 = m2c[...].T
        f1, f2 = _merge_top2(m1r[...], m2r[...], t1, t2)
        contrib = 0.5 * (jnp.sqrt(jnp.maximum(2.0 * f1, 1e-12)) +
                         jnp.sqrt(jnp.maximum(2.0 * f2, 1e-12)))

        @pl.when(r == 0)
        def _():
            acc[...] = 0.5 * contrib

        @pl.when(r == 1)
        def _():
            tot = acc[...] + 0.5 * contrib
            ps_ref[...] = tot
            sc_ref[...] = jnp.broadcast_to(jnp.max(tot, axis=1, keepdims=True),
                                           (B, 128))


def kernel(features):
    pairs = [(a, b) for a in range(B) for b in range(a + 1, B)]
    ii = jnp.asarray(np.array([a for a, _ in pairs], dtype=np.int32))
    jj = jnp.asarray(np.array([b for _, b in pairs], dtype=np.int32))

    ps, sc = pl.pallas_call(
        _fused_kernel,
        grid_spec=pltpu.PrefetchScalarGridSpec(
            num_scalar_prefetch=2,
            grid=(2, NPAIR + 1),
            in_specs=[pl.BlockSpec((B, P, D), lambda r, p, ii, jj: (0, 0, 0))],
            out_specs=[
                pl.BlockSpec((B, P), lambda r, p, ii, jj: (0, 0)),
                pl.BlockSpec((B, 128), lambda r, p, ii, jj: (0, 0)),
            ],
            scratch_shapes=[
                pltpu.VMEM((2, N, D), jnp.bfloat16),
                pltpu.VMEM((2, N, 1), jnp.float32),
                pltpu.VMEM((2, B, P), jnp.float32),
                pltpu.VMEM((P, P), jnp.float32),
                pltpu.VMEM((P, P), jnp.float32),
                pltpu.VMEM((B, P), jnp.float32),
                pltpu.VMEM((B, P), jnp.float32),
                pltpu.VMEM((P, B), jnp.float32),
                pltpu.VMEM((P, B), jnp.float32),
                pltpu.VMEM((B, P), jnp.float32),
            ],
        ),
        out_shape=[
            jax.ShapeDtypeStruct((B, P), jnp.float32),
            jax.ShapeDtypeStruct((B, 128), jnp.float32),
        ],
    )(ii, jj, features)

    scores = sc[:, 0]
    scores_pixel = ps.reshape(B, PH, PW)
    return scores, scores_pixel


# speedup vs baseline: 1.1551x; 1.1551x over previous
import jax
import jax.numpy as jnp
import numpy as np
from jax.experimental import pallas as pl
from jax.experimental.pallas import tpu as pltpu

B, PH, PW, D = 8, 16, 16, 1024
P = PH * PW
N = B * P
NPAIR = B * (B - 1) // 2


def _prep(f_ref, xs, hc, hr):
    x = f_ref[...].reshape(N, D)
    mu = jnp.mean(x, axis=-1, keepdims=True)
    var = jnp.mean((x - mu) * (x - mu), axis=-1, keepdims=True)
    xn = (x - mu) / jnp.sqrt(var + 1e-6)

    io = jax.lax.broadcasted_iota(jnp.int32, (N, 1), 0)
    pm = io % P
    ph = pm // PW
    pw = pm % PW
    z16 = jnp.zeros((16, D), jnp.float32)
    z1 = jnp.zeros((1, D), jnp.float32)
    up = jnp.concatenate([xn[16:], z16], axis=0)
    dn = jnp.concatenate([z16, xn[:-16]], axis=0)
    vs = xn + jnp.where(ph < PH - 1, up, 0.0) + jnp.where(ph > 0, dn, 0.0)
    lf = jnp.concatenate([vs[1:], z1], axis=0)
    rt = jnp.concatenate([z1, vs[:-1]], axis=0)
    hs = vs + jnp.where(pw < PW - 1, lf, 0.0) + jnp.where(pw > 0, rt, 0.0)
    cv = 3.0 - (ph == 0) - (ph == PH - 1)
    ch = 3.0 - (pw == 0) - (pw == PW - 1)
    pooled = hs / (cv * ch)

    xs[0] = xn.astype(jnp.bfloat16)
    xs[1] = pooled.astype(jnp.bfloat16)
    ones_row = jnp.ones((1, D), jnp.float32)
    for r in range(2):
        a = [xn, pooled][r]
        hc[r] = 0.5 * jnp.sum(a * a, axis=1, keepdims=True)
        for b in range(B):
            zb = a[b * P:(b + 1) * P]
            hr[r, pl.ds(b, 1)] = 0.5 * jax.lax.dot_general(
                ones_row, zb * zb, (((1,), (1,)), ((), ())),
                preferred_element_type=jnp.float32)


def _merge_top2(a1, a2, b1, b2):
    m1 = jnp.minimum(a1, b1)
    m2 = jnp.minimum(jnp.maximum(a1, b1), jnp.minimum(a2, b2))
    return m1, m2


def _step(p, ii_ref, jj_ref, r, xs, gw, gr, hc, hr, m1r, m2r, m1c, m2c):
    pm = jnp.minimum(p, NPAIR - 1)
    im = ii_ref[pm]
    jm = jj_ref[pm]
    xq = xs[r, pl.ds(im * P, P)]
    xk = xs[r, pl.ds(jm * P, P)]
    gw[...] = jax.lax.dot_general(xk, xq, (((1,), (1,)), ((), ())),
                                  preferred_element_type=jnp.float32)

    q = jnp.maximum(p - 1, 0)
    live = p > 0
    i = ii_ref[q]
    j = jj_ref[q]
    hq = hr[r, pl.ds(i, 1)]
    hk = hc[r, pl.ds(j * P, P)]
    h2 = (hk - gr[...]) + hq
    dq = jnp.min(h2, axis=0, keepdims=True)
    dk = jnp.min(h2, axis=1, keepdims=True)

    rows = jax.lax.broadcasted_iota(jnp.int32, (B, P), 0)
    urow = (rows == i) & live
    dqb = jnp.broadcast_to(dq, (B, P))
    o1 = m1r[...]
    m1r[...] = jnp.where(urow, jnp.minimum(o1, dqb), o1)
    m2r[...] = jnp.where(urow & (dqb < o1), o1,
                         jnp.where(urow, jnp.minimum(m2r[...], dqb), m2r[...]))

    cols = jax.lax.broadcasted_iota(jnp.int32, (P, B), 1)
    ucol = (cols == j) & live
    dkb = jnp.broadcast_to(dk, (P, B))
    c1 = m1c[...]
    m1c[...] = jnp.where(ucol, jnp.minimum(c1, dkb), c1)
    m2c[...] = jnp.where(ucol & (dkb < c1), c1,
                         jnp.where(ucol, jnp.minimum(m2c[...], dkb), m2c[...]))


def _fused_kernel(ii_ref, jj_ref, f_ref, ps_ref, sc_ref,
                  xs, hc, hr, ga, gb, m1r, m2r, m1c, m2c, acc):
    r = pl.program_id(0)
    p = pl.program_id(1)

    @pl.when((r == 0) & (p == 0))
    def _():
        _prep(f_ref, xs, hc, hr)

    @pl.when(p == 0)
    def _():
        m1r[...] = jnp.full((B, P), jnp.inf, jnp.float32)
        m2r[...] = jnp.full((B, P), jnp.inf, jnp.float32)
        m1c[...] = jnp.full((P, B), jnp.inf, jnp.float32)
        m2c[...] = jnp.full((P, B), jnp.inf, jnp.float32)

    even = p % 2 == 0

    @pl.when(even)
    def _():
        _step(p, ii_ref, jj_ref, r, xs, ga, gb, hc, hr, m1r, m2r, m1c, m2c)

    @pl.when(jnp.logical_not(even))
    def _():
        _step(p, ii_ref, jj_ref, r, xs, gb, ga, hc, hr, m1r, m2r, m1c, m2c)

    @pl.when(p == NPAIR)
    def _():
        t1 = m1c[...].T
        t2 = m2c[...].T
        f1, f2 = _merge_top2(m1r[...], m2r[...], t1, t2)
        contrib = 0.5 * (jnp.sqrt(jnp.maximum(2.0 * f1, 1e-12)) +
                         jnp.sqrt(jnp.maximum(2.0 * f2, 1e-12)))

        @pl.when(r == 0)
        def _():
            acc[...] = 0.5 * contrib

        @pl.when(r == 1)
        def _():
            tot = acc[...] + 0.5 * contrib
            ps_ref[...] = tot
            sc_ref[...] = jnp.broadcast_to(jnp.max(tot, axis=1, keepdims=True),
                                           (B, 128))


def kernel(features):
    pairs = [(a, b) for a in range(B) for b in range(a + 1, B)]
    ii = jnp.asarray(np.array([a for a, _ in pairs], dtype=np.int32))
    jj = jnp.asarray(np.array([b for _, b in pairs], dtype=np.int32))

    ps, sc = pl.pallas_call(
        _fused_kernel,
        grid_spec=pltpu.PrefetchScalarGridSpec(
            num_scalar_prefetch=2,
            grid=(2, NPAIR + 1),
            in_specs=[pl.BlockSpec((B, P, D), lambda r, p, ii, jj: (0, 0, 0))],
            out_specs=[
                pl.BlockSpec((B, P), lambda r, p, ii, jj: (0, 0)),
                pl.BlockSpec((B, 128), lambda r, p, ii, jj: (0, 0)),
            ],
            scratch_shapes=[
                pltpu.VMEM((2, N, D), jnp.bfloat16),
                pltpu.VMEM((2, N, 1), jnp.float32),
                pltpu.VMEM((2, B, P), jnp.float32),
                pltpu.VMEM((P, P), jnp.float32),
                pltpu.VMEM((P, P), jnp.float32),
                pltpu.VMEM((B, P), jnp.float32),
                pltpu.VMEM((B, P), jnp.float32),
                pltpu.VMEM((P, B), jnp.float32),
                pltpu.VMEM((P, B), jnp.float32),
                pltpu.VMEM((B, P), jnp.float32),
            ],
        ),
        out_shape=[
            jax.ShapeDtypeStruct((B, P), jnp.float32),
            jax.ShapeDtypeStruct((B, 128), jnp.float32),
        ],
    )(ii, jj, features)

    scores = sc[:, 0]
    scores_pixel = ps.reshape(B, PH, PW)
    return scores, scores_pixel


# speedup vs baseline: 1.2294x; 1.0643x over previous
import jax
import jax.numpy as jnp
import numpy as np
from jax.experimental import pallas as pl
from jax.experimental.pallas import tpu as pltpu

B, PH, PW, D = 8, 16, 16, 1024
P = PH * PW
N = B * P
NPAIR = B * (B - 1) // 2


def _pool_matrix():
    def mat1d(n):
        S = np.zeros((n, n), np.float32)
        for a in range(n):
            lo, hi = max(0, a - 1), min(n, a + 2)
            S[a, lo:hi] = 1.0 / (hi - lo)
        return S

    return np.kron(mat1d(PH), mat1d(PW))


def _prep(f_ref, w_ref, xs, hc, hr):
    x = f_ref[...].reshape(N, D)
    s1 = jnp.mean(x, axis=-1, keepdims=True)
    s2 = jnp.mean(x * x, axis=-1, keepdims=True)
    var = s2 - s1 * s1
    xn = (x - s1) / jnp.sqrt(var + 1e-6)
    xnb = xn.astype(jnp.bfloat16)
    xs[0] = xnb

    W = w_ref[...]
    ones_row = jnp.ones((1, D), jnp.float32)
    hc[0] = (0.5 * jnp.sum(xn * xn, axis=1, keepdims=True)).astype(jnp.bfloat16)
    for b in range(B):
        sl = pl.ds(b * P, P)
        pb = jax.lax.dot_general(W, xnb[b * P:(b + 1) * P],
                                 (((1,), (0,)), ((), ())),
                                 preferred_element_type=jnp.float32
                                 ).astype(jnp.bfloat16)
        xs[1, sl] = pb
        pf = pb.astype(jnp.float32)
        hc[1, sl] = (0.5 * jnp.sum(pf * pf, axis=1, keepdims=True)
                     ).astype(jnp.bfloat16)
        zb = xn[b * P:(b + 1) * P]
        hr[0, pl.ds(b, 1)] = 0.5 * jax.lax.dot_general(
            ones_row, zb * zb, (((1,), (1,)), ((), ())),
            preferred_element_type=jnp.float32)
        hr[1, pl.ds(b, 1)] = 0.5 * jax.lax.dot_general(
            ones_row, pf * pf, (((1,), (1,)), ((), ())),
            preferred_element_type=jnp.float32)


def _merge_top2(a1, a2, b1, b2):
    m1 = jnp.minimum(a1, b1)
    m2 = jnp.minimum(jnp.maximum(a1, b1), jnp.minimum(a2, b2))
    return m1, m2


def _step(p, ii_ref, jj_ref, r, xs, gw, gr, hc, hr, m1r, m2r, m1c, m2c):
    pm = jnp.minimum(p, NPAIR - 1)
    im = ii_ref[pm]
    jm = jj_ref[pm]
    xq = xs[r, pl.ds(im * P, P)]
    xk = xs[r, pl.ds(jm * P, P)]
    gw[...] = jax.lax.dot_general(xk, xq, (((1,), (1,)), ((), ())),
                                  preferred_element_type=jnp.float32
                                  ).astype(jnp.bfloat16)

    q = jnp.maximum(p - 1, 0)
    live = p > 0
    i = ii_ref[q]
    j = jj_ref[q]
    hq = hr[r, pl.ds(i, 1)].astype(jnp.bfloat16)
    hk = hc[r, pl.ds(j * P, P)]
    h2 = (hk - gr[...]) + hq
    dq = jnp.min(h2, axis=0, keepdims=True)
    dk = jnp.min(h2, axis=1, keepdims=True)

    rows = jax.lax.broadcasted_iota(jnp.int32, (B, P), 0).astype(jnp.bfloat16)
    urow = (rows == i.astype(jnp.bfloat16)) & live
    dqb = jnp.broadcast_to(dq, (B, P))
    o1 = m1r[...]
    m1r[...] = jnp.where(urow, jnp.minimum(o1, dqb), o1)
    m2r[...] = jnp.where(urow & (dqb < o1), o1,
                         jnp.where(urow, jnp.minimum(m2r[...], dqb), m2r[...]))

    cols = jax.lax.broadcasted_iota(jnp.int32, (P, B), 1).astype(jnp.bfloat16)
    ucol = (cols == j.astype(jnp.bfloat16)) & live
    dkb = jnp.broadcast_to(dk, (P, B))
    c1 = m1c[...]
    m1c[...] = jnp.where(ucol, jnp.minimum(c1, dkb), c1)
    m2c[...] = jnp.where(ucol & (dkb < c1), c1,
                         jnp.where(ucol, jnp.minimum(m2c[...], dkb), m2c[...]))


def _fused_kernel(ii_ref, jj_ref, f_ref, w_ref, ps_ref, sc_ref,
                  xs, hc, hr, ga, gb, m1r, m2r, m1c, m2c, acc):
    r = pl.program_id(0)
    p = pl.program_id(1)

    @pl.when((r == 0) & (p == 0))
    def _():
        _prep(f_ref, w_ref, xs, hc, hr)

    @pl.when(p == 0)
    def _():
        m1r[...] = jnp.full((B, P), jnp.inf, jnp.bfloat16)
        m2r[...] = jnp.full((B, P), jnp.inf, jnp.bfloat16)
        m1c[...] = jnp.full((P, B), jnp.inf, jnp.bfloat16)
        m2c[...] = jnp.full((P, B), jnp.inf, jnp.bfloat16)

    even = p % 2 == 0

    @pl.when(even)
    def _():
        _step(p, ii_ref, jj_ref, r, xs, ga, gb, hc, hr, m1r, m2r, m1c, m2c)

    @pl.when(jnp.logical_not(even))
    def _():
        _step(p, ii_ref, jj_ref, r, xs, gb, ga, hc, hr, m1r, m2r, m1c, m2c)

    @pl.when(p == NPAIR)
    def _():
        t1 = m1c[...].T
        t2 = m2c[...].T
        f1, f2 = _merge_top2(m1r[...], m2r[...], t1, t2)
        f1 = f1.astype(jnp.float32)
        f2 = f2.astype(jnp.float32)
        contrib = 0.5 * (jnp.sqrt(jnp.maximum(2.0 * f1, 1e-12)) +
                         jnp.sqrt(jnp.maximum(2.0 * f2, 1e-12)))

        @pl.when(r == 0)
        def _():
            acc[...] = 0.5 * contrib

        @pl.when(r == 1)
        def _():
            tot = acc[...] + 0.5 * contrib
            ps_ref[...] = tot
            sc_ref[...] = jnp.broadcast_to(jnp.max(tot, axis=1, keepdims=True),
                                           (B, 128))


def kernel(features):
    pairs = [(a, b) for a in range(B) for b in range(a + 1, B)]
    ii = jnp.asarray(np.array([a for a, _ in pairs], dtype=np.int32))
    jj = jnp.asarray(np.array([b for _, b in pairs], dtype=np.int32))
    W = jnp.asarray(_pool_matrix(), dtype=jnp.bfloat16)

    ps, sc = pl.pallas_call(
        _fused_kernel,
        grid_spec=pltpu.PrefetchScalarGridSpec(
            num_scalar_prefetch=2,
            grid=(2, NPAIR + 1),
            in_specs=[
                pl.BlockSpec((B, P, D), lambda r, p, ii, jj: (0, 0, 0)),
                pl.BlockSpec((P, P), lambda r, p, ii, jj: (0, 0)),
            ],
            out_specs=[
                pl.BlockSpec((B, P), lambda r, p, ii, jj: (0, 0)),
                pl.BlockSpec((B, 128), lambda r, p, ii, jj: (0, 0)),
            ],
            scratch_shapes=[
                pltpu.VMEM((2, N, D), jnp.bfloat16),
                pltpu.VMEM((2, N, 1), jnp.bfloat16),
                pltpu.VMEM((2, B, P), jnp.float32),
                pltpu.VMEM((P, P), jnp.bfloat16),
                pltpu.VMEM((P, P), jnp.bfloat16),
                pltpu.VMEM((B, P), jnp.bfloat16),
                pltpu.VMEM((B, P), jnp.bfloat16),
                pltpu.VMEM((P, B), jnp.bfloat16),
                pltpu.VMEM((P, B), jnp.bfloat16),
                pltpu.VMEM((B, P), jnp.float32),
            ],
        ),
        out_shape=[
            jax.ShapeDtypeStruct((B, P), jnp.float32),
            jax.ShapeDtypeStruct((B, 128), jnp.float32),
        ],
    )(ii, jj, features, W)

    scores = sc[:, 0]
    scores_pixel = ps.reshape(B, PH, PW)
    return scores, scores_pixel
